# baseline (speedup 1.0000x reference)
import functools

import jax
import jax.numpy as jnp
from jax import lax
from jax.experimental import pallas as pl
from jax.experimental.pallas import tpu as pltpu
from jax.experimental.pallas import tpu_sc as plsc

B = 4096
L = 200
D = 64
VOCAB = 1001
VPAD = 1024
NJ = 8
BLK = 512

NW = 32
TPW = B // NW
RT = 32
NR = TPW // RT
LANES = 16


def _sc_hist(subs2d):
    mesh = plsc.VectorSubcoreMesh(core_axis_name="c", subcore_axis_name="s")
    info = plsc.get_sparse_core_info()
    nc = info.num_cores
    full_chunks = L // LANES
    tail = L - full_chunks * LANES

    @functools.partial(
        pl.kernel,
        out_type=jax.ShapeDtypeStruct((B // BLK, NJ, BLK, 128), jnp.float32),
        mesh=mesh,
        scratch_types=[
            pltpu.VMEM((RT, L), jnp.int16),
            pltpu.VMEM((RT, L), jnp.int16),
            pltpu.VMEM((RT, VPAD), jnp.float32),
            pltpu.VMEM((RT, VPAD), jnp.float32),
            pltpu.SemaphoreType.DMA,
            pltpu.SemaphoreType.DMA,
            pltpu.SemaphoreType.DMA,
            pltpu.SemaphoreType.DMA,
        ],
        compiler_params=pltpu.CompilerParams(
            needs_layout_passes=False,
            use_tc_tiling_on_sc=False,
        ),
    )
    def hist_kernel(subs_hbm, c_hbm, s0, s1, h0, h1,
                    in0, in1, out0, out1):
        subs_bufs = [s0, s1]
        hist_bufs = [h0, h1]
        in_sems = [in0, in1]
        out_sems = [out0, out1]

        wid = lax.axis_index("s") * nc + lax.axis_index("c")
        ones = jnp.full((LANES,), 1.0, jnp.float32)
        zeros = jnp.zeros((LANES,), jnp.float32)
        iota = lax.iota(jnp.int32, LANES)
        tail_mask = iota >= 12

        def scatter_round(subs_v, hist_v, add):
            def put(hist_v, row, svec, mask):
                if add:
                    plsc.addupdate_scatter(hist_v, [row, svec], ones,
                                           mask=mask)
                else:
                    plsc.store_scatter(hist_v, [row, svec], zeros,
                                       mask=mask)

            @plsc.parallel_loop(0, RT, unroll=2)
            def tok_body(t):
                row = lax.broadcast(t, (LANES,))
                for c in range(L // 32 + 1):
                    start = min(c * 32, L - 32)
                    v32 = plsc.bitcast(
                        subs_v[t, pl.ds(start, 32)], jnp.int32
                    )
                    lo = v32 & 0xFFFF
                    hi = lax.shift_right_logical(v32, 16)
                    m = None if c < L // 32 else tail_mask
                    put(hist_v, row, lo, m)
                    put(hist_v, row, hi, m)

        def start_subs_in(r, bi):
            tok_base = wid * TPW + r * RT
            return pltpu.async_copy(
                subs_hbm.at[pl.ds(tok_base, RT), :],
                subs_bufs[bi], in_sems[bi],
            )

        def start_hist_out(r, bi):
            tok_base = wid * TPW + r * RT
            blk_id = tok_base // BLK
            row_off = tok_base % BLK
            copies = []
            for j in range(NJ):
                copies.append(pltpu.async_copy(
                    hist_bufs[bi].at[:, pl.ds(j * 128, 128)],
                    c_hbm.at[blk_id, j, pl.ds(row_off, RT), :],
                    out_sems[bi],
                ))
            return copies

        in_flight = [None, None]
        out_flight = [None, None]
        in_flight[0] = start_subs_in(0, 0)
        in_flight[1] = start_subs_in(1, 1)

        @plsc.parallel_loop(0, RT, unroll=2)
        def zero_body(i):
            for j in range(VPAD // LANES):
                h0[i, pl.ds(j * LANES, LANES)] = zeros
                h1[i, pl.ds(j * LANES, LANES)] = zeros

        for r in range(NR):
            bi = r % 2
            in_flight[bi].wait()
            scatter_round(subs_bufs[bi], hist_bufs[bi], add=True)
            out_flight[bi] = start_hist_out(r, bi)
            if r >= 1:
                ob = 1 - bi
                for cp in out_flight[ob]:
                    cp.wait()
                if r + 1 < NR:
                    scatter_round(subs_bufs[ob], hist_bufs[ob], add=False)
                    in_flight[ob] = start_subs_in(r + 1, ob)
        for cp in out_flight[(NR - 1) % 2]:
            cp.wait()

    return hist_kernel(subs2d)


def _tc_pool(hist4, tbl4):
    def body(c_ref, t_ref, o_ref):
        acc = jnp.dot(
            c_ref[0, 0], t_ref[0], preferred_element_type=jnp.float32
        )
        for j in range(1, NJ):
            acc += jnp.dot(
                c_ref[0, j], t_ref[j], preferred_element_type=jnp.float32
            )
        cnt = float(L) - c_ref[0, 0, :, 0:1]
        o_ref[...] = acc / cnt

    return pl.pallas_call(
        body,
        grid=(B // BLK,),
        in_specs=[
            pl.BlockSpec((1, NJ, BLK, 128), lambda i: (i, 0, 0, 0)),
            pl.BlockSpec((NJ, 128, D), lambda i: (0, 0, 0)),
        ],
        out_specs=pl.BlockSpec((BLK, D), lambda i: (i, 0)),
        out_shape=jax.ShapeDtypeStruct((B, D), jnp.float32),
    )(hist4, tbl4)


def kernel(subs, table):
    hist4 = _sc_hist(subs.astype(jnp.int16))
    tbl_pad = jnp.zeros((VPAD, D), jnp.float32).at[:VOCAB].set(table)
    return _tc_pool(hist4, jnp.reshape(tbl_pad, (NJ, 128, D)))


# speedup vs baseline: 1.0659x; 1.0659x over previous
import functools

import jax
import jax.numpy as jnp
from jax import lax
from jax.experimental import pallas as pl
from jax.experimental.pallas import tpu as pltpu
from jax.experimental.pallas import tpu_sc as plsc

B = 4096
L = 200
D = 64
VOCAB = 1001
VPAD = 1024
NJ = 8
BLK = 1024

NW = 32
TPW = B // NW
RT = 32
NR = TPW // RT
LANES = 16


def _sc_hist(subs2d):
    mesh = plsc.VectorSubcoreMesh(core_axis_name="c", subcore_axis_name="s")
    info = plsc.get_sparse_core_info()
    nc = info.num_cores
    full_chunks = L // LANES
    tail = L - full_chunks * LANES

    @functools.partial(
        pl.kernel,
        out_type=jax.ShapeDtypeStruct((B // BLK, NJ, BLK, 128), jnp.float32),
        mesh=mesh,
        scratch_types=[
            pltpu.VMEM((RT, L), jnp.int32),
            pltpu.VMEM((RT, L), jnp.int32),
            pltpu.VMEM((RT, VPAD), jnp.float32),
            pltpu.VMEM((RT, VPAD), jnp.float32),
            pltpu.SemaphoreType.DMA,
            pltpu.SemaphoreType.DMA,
            pltpu.SemaphoreType.DMA,
            pltpu.SemaphoreType.DMA,
        ],
        compiler_params=pltpu.CompilerParams(
            needs_layout_passes=False,
            use_tc_tiling_on_sc=False,
        ),
    )
    def hist_kernel(subs_hbm, c_hbm, s0, s1, h0, h1,
                    in0, in1, out0, out1):
        subs_bufs = [s0, s1]
        hist_bufs = [h0, h1]
        in_sems = [in0, in1]
        out_sems = [out0, out1]

        wid = lax.axis_index("s") * nc + lax.axis_index("c")
        ones = jnp.full((LANES,), 1.0, jnp.float32)
        zeros = jnp.zeros((LANES,), jnp.float32)
        iota = lax.iota(jnp.int32, LANES)
        tail_mask = iota >= (LANES - tail)

        def scatter_round(subs_v, hist_v, add):
            @plsc.parallel_loop(0, RT, unroll=2)
            def tok_body(t):
                row = lax.broadcast(t, (LANES,))
                for c in range(full_chunks):
                    svec = subs_v[t, pl.ds(c * LANES, LANES)]
                    if add:
                        plsc.addupdate_scatter(hist_v, [row, svec], ones)
                    else:
                        plsc.store_scatter(hist_v, [row, svec], zeros)
                svec = subs_v[t, pl.ds(L - LANES, LANES)]
                if add:
                    plsc.addupdate_scatter(
                        hist_v, [row, svec], ones, mask=tail_mask
                    )
                else:
                    plsc.store_scatter(
                        hist_v, [row, svec], zeros, mask=tail_mask
                    )

        def start_subs_in(r, bi):
            tok_base = wid * TPW + r * RT
            return pltpu.async_copy(
                subs_hbm.at[pl.ds(tok_base, RT), :],
                subs_bufs[bi], in_sems[bi],
            )

        def start_hist_out(r, bi):
            tok_base = wid * TPW + r * RT
            blk_id = tok_base // BLK
            row_off = tok_base % BLK
            copies = []
            for j in range(NJ):
                copies.append(pltpu.async_copy(
                    hist_bufs[bi].at[:, pl.ds(j * 128, 128)],
                    c_hbm.at[blk_id, j, pl.ds(row_off, RT), :],
                    out_sems[bi],
                ))
            return copies

        in_flight = [None, None]
        out_flight = [None, None]
        in_flight[0] = start_subs_in(0, 0)
        in_flight[1] = start_subs_in(1, 1)

        @plsc.parallel_loop(0, RT, unroll=2)
        def zero_body(i):
            for j in range(VPAD // LANES):
                h0[i, pl.ds(j * LANES, LANES)] = zeros
                h1[i, pl.ds(j * LANES, LANES)] = zeros

        for r in range(NR):
            bi = r % 2
            in_flight[bi].wait()
            scatter_round(subs_bufs[bi], hist_bufs[bi], add=True)
            out_flight[bi] = start_hist_out(r, bi)
            if r >= 1:
                ob = 1 - bi
                for cp in out_flight[ob]:
                    cp.wait()
                if r + 1 < NR:
                    scatter_round(subs_bufs[ob], hist_bufs[ob], add=False)
                    in_flight[ob] = start_subs_in(r + 1, ob)
        for cp in out_flight[(NR - 1) % 2]:
            cp.wait()

    return hist_kernel(subs2d)


def _tc_pool(hist4, tbl4):
    def body(c_ref, t_ref, o_ref):
        acc = jnp.dot(
            c_ref[0, 0], t_ref[0], preferred_element_type=jnp.float32
        )
        for j in range(1, NJ):
            acc += jnp.dot(
                c_ref[0, j], t_ref[j], preferred_element_type=jnp.float32
            )
        cnt = float(L) - c_ref[0, 0, :, 0:1]
        o_ref[...] = acc / cnt

    return pl.pallas_call(
        body,
        grid=(B // BLK,),
        in_specs=[
            pl.BlockSpec((1, NJ, BLK, 128), lambda i: (i, 0, 0, 0)),
            pl.BlockSpec((NJ, 128, D), lambda i: (0, 0, 0)),
        ],
        out_specs=pl.BlockSpec((BLK, D), lambda i: (i, 0)),
        out_shape=jax.ShapeDtypeStruct((B, D), jnp.float32),
    )(hist4, tbl4)


def kernel(subs, table):
    hist4 = _sc_hist(subs.astype(jnp.int32))
    tbl_pad = jnp.zeros((VPAD, D), jnp.float32).at[:VOCAB].set(table)
    return _tc_pool(hist4, jnp.reshape(tbl_pad, (NJ, 128, D)))
